# baseline (speedup 1.0000x reference)
import functools

import jax
import jax.numpy as jnp
from jax import lax
from jax.experimental import pallas as pl
from jax.experimental.pallas import tpu as pltpu
from jax.experimental.pallas import tpu_sc as plsc

T = 2048
NHID = 1024
NCLS = 100
CHUNK = 1000
P = 20
PPAD = 32

NW = 32
CLS_PER_W = 4


def _sc_gather(x, idx_pad):
    mesh = plsc.VectorSubcoreMesh(core_axis_name="c", subcore_axis_name="s")

    @functools.partial(
        pl.kernel,
        out_type=jax.ShapeDtypeStruct((NCLS, PPAD, NHID), jnp.float32),
        mesh=mesh,
        scratch_types=[
            pltpu.VMEM((PPAD,), jnp.int32),
            pltpu.VMEM((PPAD, NHID), jnp.float32),
            pltpu.SemaphoreType.DMA,
        ],
    )
    def gather_k(x_hbm, idx_hbm, out_hbm, idx_v, rows_v, sem):
        wid = lax.axis_index("s") * 2 + lax.axis_index("c")
        for k in range(CLS_PER_W):
            cls = wid * CLS_PER_W + k

            @pl.when(cls < NCLS)
            def _():
                pltpu.sync_copy(idx_hbm.at[cls], idx_v)
                pltpu.async_copy(x_hbm.at[idx_v], rows_v, sem).wait()
                pltpu.sync_copy(rows_v, out_hbm.at[cls])

    return gather_k(x, idx_pad)


def _tc_body(x_ref, d_ref, Wc_ref, bc_ref, Ww_ref, bw_ref,
             pclass_ref, pwords_ref):
    c = pl.program_id(0)

    @pl.when(c == 0)
    def _():
        pc = lax.dot_general(x_ref[...], Wc_ref[...],
                             (((1,), (1,)), ((), ())),
                             preferred_element_type=jnp.float32)
        pclass_ref[...] = pc + bc_ref[...]

    d = d_ref[0, :P, :]
    w = Ww_ref[0]
    pw = lax.dot_general(d, w, (((1,), (1,)), ((), ())),
                         preferred_element_type=jnp.float32)
    pwords_ref[0] = pw + bw_ref[0]


def _tc_decode(x, d_pad, Wc, bc2, Ww, bw):
    return pl.pallas_call(
        _tc_body,
        grid=(NCLS,),
        in_specs=[
            pl.BlockSpec((T, NHID), lambda c: (0, 0)),
            pl.BlockSpec((1, PPAD, NHID), lambda c: (c, 0, 0)),
            pl.BlockSpec((NCLS, NHID), lambda c: (0, 0)),
            pl.BlockSpec((1, NCLS), lambda c: (0, 0)),
            pl.BlockSpec((1, CHUNK, NHID), lambda c: (c, 0, 0)),
            pl.BlockSpec((1, 1, CHUNK), lambda c: (c, 0, 0)),
        ],
        out_specs=[
            pl.BlockSpec((T, NCLS), lambda c: (0, 0)),
            pl.BlockSpec((1, P, CHUNK), lambda c: (c, 0, 0)),
        ],
        out_shape=[
            jax.ShapeDtypeStruct((T, NCLS), jnp.float32),
            jax.ShapeDtypeStruct((NCLS, P, CHUNK), jnp.float32),
        ],
    )(x, d_pad, Wc, bc2, Ww, bw.reshape(NCLS, 1, CHUNK))


def kernel(x, within_batch_idx, Wc, bc, Ww, bw):
    idx32 = within_batch_idx.astype(jnp.int32)
    idx_pad = jnp.pad(idx32, ((0, 0), (0, PPAD - P)))
    d_pad = _sc_gather(x, idx_pad)
    p_class, p_words = _tc_decode(x, d_pad, Wc, bc.reshape(1, NCLS), Ww, bw)
    return (p_class, p_words)
